# Initial kernel scaffold; baseline (speedup 1.0000x reference)
import jax
import jax.numpy as jnp
from jax.experimental import pallas as pl


def kernel(x, indptr):
    raise NotImplementedError("write your pallas kernel here")


# speedup vs baseline: 12.0427x; 12.0427x over previous
import jax
import jax.numpy as jnp
from jax import lax
from jax.experimental import pallas as pl
from jax.experimental.pallas import tpu as pltpu
from jax.experimental.pallas import tpu_sc as plsc

N_SEG = 10000
E = 320000
D = 128
NV = D // 16

NW = 32
SEG_A = 320
SEG_B = 312
CHUNK = 256
ILEN = 336
IPTR_PAD = 10368


def _body(x_hbm, iptr_hbm, out_hbm, iptr_v, rowbuf, outbuf):
    cid = lax.axis_index("c")
    sid = lax.axis_index("s")
    w = sid * 2 + cid
    seg_lo = w * SEG_B + 8 * jnp.minimum(w, 2)
    nseg = jnp.where(w < 2, SEG_A, SEG_B)
    pltpu.sync_copy(iptr_hbm.at[pl.ds(seg_lo, ILEN)], iptr_v)

    zeros = tuple(jnp.zeros((16,), jnp.float32) for _ in range(NV))

    def seg_body(s, chunk_start):
        bounds = iptr_v[pl.ds(s, 16)]
        lo = bounds[0]
        hi = bounds[1]

        def row_body(r, carry):
            cs = carry[0]
            accs = carry[1:]

            def load():
                ns = jnp.minimum((r // 8) * 8, E - CHUNK)
                pltpu.sync_copy(x_hbm.at[pl.ds(ns, CHUNK)], rowbuf)
                return ns

            cs = lax.cond(r >= cs + CHUNK, load, lambda: cs)
            o = r - cs
            accs = tuple(accs[j] + rowbuf[o, pl.ds(j * 16, 16)]
                         for j in range(NV))
            return (cs,) + accs

        res = lax.fori_loop(lo, hi, row_body, (chunk_start,) + zeros)
        n = hi - lo
        nvec = jnp.full((16,), 1.0, jnp.float32) * (
            jnp.maximum(n, 1).astype(jnp.float32))
        scale = jnp.full((16,), 1.0, jnp.float32) / nvec
        for j in range(NV):
            outbuf[s, pl.ds(j * 16, 16)] = res[1 + j] * scale
        return res[0]

    lax.fori_loop(0, nseg, seg_body, jnp.int32(-CHUNK))

    @pl.when(w < 2)
    def _():
        pltpu.sync_copy(outbuf.at[pl.ds(0, SEG_A)],
                        out_hbm.at[pl.ds(seg_lo, SEG_A)])

    @pl.when(w >= 2)
    def _():
        pltpu.sync_copy(outbuf.at[pl.ds(0, SEG_B)],
                        out_hbm.at[pl.ds(seg_lo, SEG_B)])


def kernel(x, indptr):
    iptr = jnp.concatenate([
        indptr.astype(jnp.int32),
        jnp.full((IPTR_PAD - (N_SEG + 1),), E, jnp.int32),
    ])
    mesh = plsc.VectorSubcoreMesh(core_axis_name="c", subcore_axis_name="s")
    f = pl.kernel(
        _body,
        mesh=mesh,
        out_type=jax.ShapeDtypeStruct((N_SEG, D), jnp.float32),
        scratch_types=[
            pltpu.VMEM((ILEN,), jnp.int32),
            pltpu.VMEM((CHUNK, D), jnp.float32),
            pltpu.VMEM((SEG_A, D), jnp.float32),
        ],
    )
    return f(x, iptr)


# speedup vs baseline: 25.1517x; 2.0885x over previous
import jax
import jax.numpy as jnp
from jax import lax
from jax.experimental import pallas as pl
from jax.experimental.pallas import tpu as pltpu
from jax.experimental.pallas import tpu_sc as plsc

N_SEG = 10000
E = 320000
D = 128
NV = D // 16

NW = 32
SEG_A = 320
SEG_B = 312
CHUNK = 256
ILEN = 336
IPTR_PAD = 10368


def _body(x_hbm, iptr_hbm, out_hbm, iptr_v, rowbuf, outbuf, sems):
    cid = lax.axis_index("c")
    sid = lax.axis_index("s")
    w = sid * 2 + cid
    seg_lo = w * SEG_B + 8 * jnp.minimum(w, 2)
    nseg = jnp.where(w < 2, SEG_A, SEG_B)
    pltpu.sync_copy(iptr_hbm.at[pl.ds(seg_lo, ILEN)], iptr_v)

    row_lo = iptr_v[pl.ds(0, 16)][0]
    row_hi = iptr_v[pl.ds(nseg, 16)][0]
    start0 = jnp.minimum((row_lo // 8) * 8, E - CHUNK)
    nchunks = jnp.maximum((row_hi - start0 + CHUNK - 1) // CHUNK, 1)

    def chunk_start(c):
        return jnp.minimum(start0 + c * CHUNK, E - CHUNK)

    def dma(c):
        b = lax.rem(c, 2)
        return pltpu.make_async_copy(
            x_hbm.at[pl.ds(chunk_start(c), CHUNK)], rowbuf.at[b], sems.at[b])

    dma(jnp.int32(0)).start()

    @pl.when(nchunks > 1)
    def _():
        dma(jnp.int32(1)).start()

    dma(jnp.int32(0)).wait()

    zeros = tuple(jnp.zeros((16,), jnp.float32) for _ in range(NV))

    nsteps = nseg + nchunks - 1

    def step(_, st):
        r, s_cur, c = st[0], st[1], st[2]
        accs = st[3:]
        bounds = iptr_v[pl.ds(s_cur, 16)]
        lo = bounds[0]
        hi = bounds[1]
        r_end = jnp.minimum(row_hi, start0 + (c + 1) * CHUNK)
        run_end = jnp.minimum(hi, r_end)
        b = lax.rem(c, 2)
        sc = chunk_start(c)

        def row_body(rr, a):
            o = rr - sc
            return tuple(a[j] + rowbuf[b, o, pl.ds(j * 16, 16)]
                         for j in range(NV))

        accs = lax.fori_loop(r, run_end, row_body, accs)

        fin = hi <= r_end

        @pl.when(fin)
        def _():
            nvec = jnp.full((16,), 1.0, jnp.float32) * (
                jnp.maximum(hi - lo, 1).astype(jnp.float32))
            scale = jnp.full((16,), 1.0, jnp.float32) / nvec
            for j in range(NV):
                outbuf[s_cur, pl.ds(j * 16, 16)] = accs[j] * scale

        @pl.when(jnp.logical_not(fin))
        def _():
            dma(c + 1).wait()

            @pl.when(c + 2 < nchunks)
            def _():
                dma(c + 2).start()

        s_next = jnp.where(fin, s_cur + 1, s_cur)
        c_next = jnp.where(fin, c, c + 1)
        accs = tuple(
            jnp.where(fin, jnp.zeros((16,), jnp.float32), a)
            for a in accs)
        return (run_end, s_next, c_next) + accs

    lax.fori_loop(0, nsteps, step,
                  (row_lo, jnp.int32(0), jnp.int32(0)) + zeros)

    @pl.when(w < 2)
    def _():
        pltpu.sync_copy(outbuf.at[pl.ds(0, SEG_A)],
                        out_hbm.at[pl.ds(seg_lo, SEG_A)])

    @pl.when(w >= 2)
    def _():
        pltpu.sync_copy(outbuf.at[pl.ds(0, SEG_B)],
                        out_hbm.at[pl.ds(seg_lo, SEG_B)])


def kernel(x, indptr):
    iptr = jnp.concatenate([
        indptr.astype(jnp.int32),
        jnp.full((IPTR_PAD - (N_SEG + 1),), E, jnp.int32),
    ])
    mesh = plsc.VectorSubcoreMesh(core_axis_name="c", subcore_axis_name="s")
    f = pl.kernel(
        _body,
        mesh=mesh,
        out_type=jax.ShapeDtypeStruct((N_SEG, D), jnp.float32),
        scratch_types=[
            pltpu.VMEM((ILEN,), jnp.int32),
            pltpu.VMEM((2, CHUNK, D), jnp.float32),
            pltpu.VMEM((SEG_A, D), jnp.float32),
            pltpu.SemaphoreType.DMA((2,)),
        ],
    )
    return f(x, iptr)


# speedup vs baseline: 26.2763x; 1.0447x over previous
import jax
import jax.numpy as jnp
from jax import lax
from jax.experimental import pallas as pl
from jax.experimental.pallas import tpu as pltpu
from jax.experimental.pallas import tpu_sc as plsc

N_SEG = 10000
E = 320000
D = 128
NV = D // 16

NW = 32
SEG_A = 320
SEG_B = 312
CHUNK = 320
ILEN = 336
IPTR_PAD = 10368


def _body(x_hbm, iptr_hbm, out_hbm, iptr_v, rowbuf, outbuf, sems):
    cid = lax.axis_index("c")
    sid = lax.axis_index("s")
    w = sid * 2 + cid
    seg_lo = w * SEG_B + 8 * jnp.minimum(w, 2)
    nseg = jnp.where(w < 2, SEG_A, SEG_B)
    pltpu.sync_copy(iptr_hbm.at[pl.ds(seg_lo, ILEN)], iptr_v)

    row_lo = iptr_v[pl.ds(0, 16)][0]
    row_hi = iptr_v[pl.ds(nseg, 16)][0]
    start0 = jnp.minimum((row_lo // 8) * 8, E - CHUNK)
    nchunks = jnp.maximum((row_hi - start0 + CHUNK - 1) // CHUNK, 1)

    def chunk_start(c):
        return jnp.minimum(start0 + c * CHUNK, E - CHUNK)

    def dma(c):
        b = lax.rem(c, 2)
        return pltpu.make_async_copy(
            x_hbm.at[pl.ds(chunk_start(c), CHUNK)], rowbuf.at[b], sems.at[b])

    dma(jnp.int32(0)).start()

    @pl.when(nchunks > 1)
    def _():
        dma(jnp.int32(1)).start()

    dma(jnp.int32(0)).wait()

    zeros = tuple(jnp.zeros((16,), jnp.float32) for _ in range(NV))

    nsteps = nseg + nchunks - 1

    def step(_, st):
        r, s_cur, c = st[0], st[1], st[2]
        accs = st[3:]
        bounds = iptr_v[pl.ds(s_cur, 16)]
        lo = bounds[0]
        hi = bounds[1]
        r_end = jnp.minimum(row_hi, start0 + (c + 1) * CHUNK)
        run_end = jnp.minimum(hi, r_end)
        b = lax.rem(c, 2)
        sc = chunk_start(c)

        def row_body(rr, a):
            o = rr - sc
            return tuple(a[j] + rowbuf[b, o, pl.ds(j * 16, 16)]
                         for j in range(NV))

        accs = lax.fori_loop(r, run_end, row_body, accs)

        fin = hi <= r_end

        @pl.when(fin)
        def _():
            nvec = jnp.full((16,), 1.0, jnp.float32) * (
                jnp.maximum(hi - lo, 1).astype(jnp.float32))
            scale = jnp.full((16,), 1.0, jnp.float32) / nvec
            for j in range(NV):
                outbuf[s_cur, pl.ds(j * 16, 16)] = accs[j] * scale

        @pl.when(jnp.logical_not(fin))
        def _():
            dma(c + 1).wait()

            @pl.when(c + 2 < nchunks)
            def _():
                dma(c + 2).start()

        s_next = jnp.where(fin, s_cur + 1, s_cur)
        c_next = jnp.where(fin, c, c + 1)
        accs = tuple(
            jnp.where(fin, jnp.zeros((16,), jnp.float32), a)
            for a in accs)
        return (run_end, s_next, c_next) + accs

    lax.fori_loop(0, nsteps, step,
                  (row_lo, jnp.int32(0), jnp.int32(0)) + zeros)

    @pl.when(w < 2)
    def _():
        pltpu.sync_copy(outbuf.at[pl.ds(0, SEG_A)],
                        out_hbm.at[pl.ds(seg_lo, SEG_A)])

    @pl.when(w >= 2)
    def _():
        pltpu.sync_copy(outbuf.at[pl.ds(0, SEG_B)],
                        out_hbm.at[pl.ds(seg_lo, SEG_B)])


def kernel(x, indptr):
    iptr = jnp.concatenate([
        indptr.astype(jnp.int32),
        jnp.full((IPTR_PAD - (N_SEG + 1),), E, jnp.int32),
    ])
    mesh = plsc.VectorSubcoreMesh(core_axis_name="c", subcore_axis_name="s")
    f = pl.kernel(
        _body,
        mesh=mesh,
        out_type=jax.ShapeDtypeStruct((N_SEG, D), jnp.float32),
        scratch_types=[
            pltpu.VMEM((ILEN,), jnp.int32),
            pltpu.VMEM((2, CHUNK, D), jnp.float32),
            pltpu.VMEM((SEG_A, D), jnp.float32),
            pltpu.SemaphoreType.DMA((2,)),
        ],
    )
    return f(x, iptr)


# speedup vs baseline: 26.3604x; 1.0032x over previous
import jax
import jax.numpy as jnp
from jax import lax
from jax.experimental import pallas as pl
from jax.experimental.pallas import tpu as pltpu
from jax.experimental.pallas import tpu_sc as plsc

N_SEG = 10000
E = 320000
D = 128
NV = D // 16

NW = 32
SEG_A = 320
SEG_B = 312
CHUNK = 320
ILEN = 336
IPTR_PAD = 10368


def _body(x_hbm, iptr_hbm, out_hbm, iptr_v, rowbuf, outbuf, sems):
    cid = lax.axis_index("c")
    sid = lax.axis_index("s")
    w = sid * 2 + cid
    seg_lo = w * SEG_B + 8 * jnp.minimum(w, 2)
    nseg = jnp.where(w < 2, SEG_A, SEG_B)
    pltpu.sync_copy(iptr_hbm.at[pl.ds(seg_lo, ILEN)], iptr_v)

    row_lo = iptr_v[pl.ds(0, 16)][0]
    row_hi = iptr_v[pl.ds(nseg, 16)][0]
    start0 = jnp.minimum((row_lo // 8) * 8, E - CHUNK)
    nchunks = jnp.maximum((row_hi - start0 + CHUNK - 1) // CHUNK, 1)

    def chunk_start(c):
        return jnp.minimum(start0 + c * CHUNK, E - CHUNK)

    def dma(c):
        b = lax.rem(c, 2)
        return pltpu.make_async_copy(
            x_hbm.at[pl.ds(chunk_start(c), CHUNK)], rowbuf.at[b], sems.at[b])

    dma(jnp.int32(0)).start()

    @pl.when(nchunks > 1)
    def _():
        dma(jnp.int32(1)).start()

    dma(jnp.int32(0)).wait()

    zeros = tuple(jnp.zeros((16,), jnp.float32) for _ in range(NV))

    nsteps = nseg + nchunks - 1

    def step(_, st):
        r, s_cur, c = st[0], st[1], st[2]
        accs = st[3:]
        bounds = iptr_v[pl.ds(s_cur, 16)]
        lo = bounds[0]
        hi = bounds[1]
        r_end = jnp.minimum(row_hi, start0 + (c + 1) * CHUNK)
        run_end = jnp.minimum(hi, r_end)
        b = lax.rem(c, 2)
        sc = chunk_start(c)

        def row_body(rr, a):
            o = rr - sc
            return tuple(a[j] + rowbuf[b, o, pl.ds(j * 16, 16)]
                         for j in range(NV))

        accs = plsc.parallel_loop(r, run_end, unroll=4,
                                  carry=accs)(row_body)

        fin = hi <= r_end

        @pl.when(fin)
        def _():
            nvec = jnp.full((16,), 1.0, jnp.float32) * (
                jnp.maximum(hi - lo, 1).astype(jnp.float32))
            scale = jnp.full((16,), 1.0, jnp.float32) / nvec
            for j in range(NV):
                outbuf[s_cur, pl.ds(j * 16, 16)] = accs[j] * scale

        @pl.when(jnp.logical_not(fin))
        def _():
            dma(c + 1).wait()

            @pl.when(c + 2 < nchunks)
            def _():
                dma(c + 2).start()

        s_next = jnp.where(fin, s_cur + 1, s_cur)
        c_next = jnp.where(fin, c, c + 1)
        accs = tuple(
            jnp.where(fin, jnp.zeros((16,), jnp.float32), a)
            for a in accs)
        return (run_end, s_next, c_next) + accs

    lax.fori_loop(0, nsteps, step,
                  (row_lo, jnp.int32(0), jnp.int32(0)) + zeros)

    @pl.when(w < 2)
    def _():
        pltpu.sync_copy(outbuf.at[pl.ds(0, SEG_A)],
                        out_hbm.at[pl.ds(seg_lo, SEG_A)])

    @pl.when(w >= 2)
    def _():
        pltpu.sync_copy(outbuf.at[pl.ds(0, SEG_B)],
                        out_hbm.at[pl.ds(seg_lo, SEG_B)])


def kernel(x, indptr):
    iptr = jnp.concatenate([
        indptr.astype(jnp.int32),
        jnp.full((IPTR_PAD - (N_SEG + 1),), E, jnp.int32),
    ])
    mesh = plsc.VectorSubcoreMesh(core_axis_name="c", subcore_axis_name="s")
    f = pl.kernel(
        _body,
        mesh=mesh,
        out_type=jax.ShapeDtypeStruct((N_SEG, D), jnp.float32),
        scratch_types=[
            pltpu.VMEM((ILEN,), jnp.int32),
            pltpu.VMEM((2, CHUNK, D), jnp.float32),
            pltpu.VMEM((SEG_A, D), jnp.float32),
            pltpu.SemaphoreType.DMA((2,)),
        ],
    )
    return f(x, iptr)
